# baseline (speedup 1.0000x reference)
import functools

import jax
import jax.numpy as jnp
from jax import lax
from jax.experimental import pallas as pl
from jax.experimental.pallas import tpu as pltpu
from jax.experimental.pallas import tpu_sc as plsc

N = 100000
NP = 100096
E = 3200000
H = 128
OUT = 3
BN_EPS = 1e-5
BN = 3128
GRID = NP // BN

NC, NS = 2, 16
CHN = 12512
ACC = 12544
DUMP = 12512
NPA = NP
BE = 2000
NBLK = (E // NS) // BE
FCAP = BE + 80
_MESH = plsc.VectorSubcoreMesh(core_axis_name="c", subcore_axis_name="s")


def _sc_agg_body(h_hbm, src_hbm, dst_hbm, agg_hbm,
                 acc, sbufA, sbufB, dbufA, dbufB, csrcf, cdstf,
                 g32sA, g32sB, g32d, rowsA, rowsB, isem, gsem):
    sbufs, dbufs = (sbufA, sbufB), (dbufA, dbufB)
    g32ss, rowss = (g32sA, g32sB), (rowsA, rowsB)
    c = lax.axis_index("c")
    s = lax.axis_index("s")

    ept = E // NS
    ebase = s * ept
    zv = jnp.zeros((16,), jnp.int32)

    @pl.loop(0, FCAP // 16)
    def _ci(i):
        csrcf[pl.ds(i * 16, 16)] = zv
        cdstf[pl.ds(i * 16, 16)] = zv + DUMP

    for j in range(4):
        chunk = 2 * j + c
        lo = chunk * CHN

        @pl.loop(0, 32)
        def _zr(i):
            for q in range(8):
                rowsA[i, pl.ds(q * 16, 16)] = jnp.zeros((16,), jnp.float32)

        @pl.when(s < 8)
        def _():
            for k in range(49):
                pltpu.async_copy(rowsA,
                                 acc.at[pl.ds(s * 1568 + k * 32, 32)], gsem)
            for k in range(49):
                pltpu.make_async_copy(rowsA,
                                      acc.at[pl.ds(s * 1568 + k * 32, 32)],
                                      gsem).wait()
        plsc.subcore_barrier()

        pltpu.async_copy(src_hbm.at[pl.ds(ebase, BE)], sbufA, isem)
        pltpu.async_copy(dst_hbm.at[pl.ds(ebase, BE)], dbufA, isem)

        @pl.loop(0, NBLK // 2)
        def _blk2(i2):
            for b in range(2):
                i = i2 * 2 + b
                nb = 1 - b
                pltpu.make_async_copy(src_hbm.at[pl.ds(ebase, BE)],
                                      sbufs[b], isem).wait()
                pltpu.make_async_copy(dst_hbm.at[pl.ds(ebase, BE)],
                                      dbufs[b], isem).wait()
                off_n = ebase + jnp.minimum(i + 1, NBLK - 1) * BE
                pltpu.async_copy(src_hbm.at[pl.ds(off_n, BE)],
                                 sbufs[nb], isem)
                pltpu.async_copy(dst_hbm.at[pl.ds(off_n, BE)],
                                 dbufs[nb], isem)

                @pl.loop(0, BE // 16, init_carry=jnp.int32(0))
                def _vec(v, cur):
                    sv = sbufs[b][pl.ds(v * 16, 16)]
                    dv = dbufs[b][pl.ds(v * 16, 16)]
                    m = (dv >= lo) & (dv < lo + CHN)
                    plsc.store_compressed(csrcf.at[pl.ds(cur, 16)], sv, mask=m)
                    plsc.store_compressed(cdstf.at[pl.ds(cur, 16)], dv - lo,
                                          mask=m)
                    return cur + jnp.sum(jnp.where(m, 1, 0).astype(jnp.int32))

                cur = _vec
                for t in range(4):
                    csrcf[pl.ds(cur + 16 * t, 16)] = zv
                    cdstf[pl.ds(cur + 16 * t, 16)] = zv + DUMP

                ng2 = (cur + 63) // 64
                for t in range(2):
                    g32sA[pl.ds(t * 16, 16)] = csrcf[pl.ds(t * 16, 16)]
                pltpu.async_copy(h_hbm.at[g32sA], rowsA, gsem)

                @pl.loop(0, ng2)
                def _fl(q):
                    for t in range(2):
                        goff = q * 64 + t * 32
                        noff = jnp.minimum(goff + 32, FCAP - 32)
                        nt = 1 - t
                        for tt in range(2):
                            g32ss[nt][pl.ds(tt * 16, 16)] = (
                                csrcf[pl.ds(noff + tt * 16, 16)])
                        pltpu.make_async_copy(h_hbm.at[g32ss[t]],
                                              rowss[t], gsem).wait()
                        pltpu.async_copy(h_hbm.at[g32ss[nt]],
                                         rowss[nt], gsem)
                        for tt in range(2):
                            g32d[pl.ds(tt * 16, 16)] = (
                                cdstf[pl.ds(goff + tt * 16, 16)])
                        pltpu.sync_copy(rowss[t], acc.at[g32d], add=True)

                pltpu.make_async_copy(h_hbm.at[g32sA],
                                      rowsA, gsem).wait()

        pltpu.make_async_copy(src_hbm.at[pl.ds(ebase, BE)],
                              sbufA, isem).wait()
        pltpu.make_async_copy(dst_hbm.at[pl.ds(ebase, BE)],
                              dbufA, isem).wait()

        plsc.subcore_barrier()

        @pl.when(s < 7)
        def _():
            pltpu.sync_copy(acc.at[pl.ds(s * 1568, 1568)],
                            agg_hbm.at[pl.ds(lo + s * 1568, 1568)])
        @pl.when(s == 7)
        def _():
            pltpu.sync_copy(acc.at[pl.ds(7 * 1568, 1536)],
                            agg_hbm.at[pl.ds(lo + 7 * 1568, 1536)])
        plsc.subcore_barrier()


def _agg128(h, src, dst):
    return pl.kernel(
        _sc_agg_body,
        out_type=jax.ShapeDtypeStruct((NPA, H), jnp.float32),
        mesh=_MESH,
        scratch_types=[
            pltpu.VMEM_SHARED((ACC, H), jnp.float32),
            pltpu.VMEM((BE,), jnp.int32),
            pltpu.VMEM((BE,), jnp.int32),
            pltpu.VMEM((BE,), jnp.int32),
            pltpu.VMEM((BE,), jnp.int32),
            pltpu.VMEM((FCAP,), jnp.int32),
            pltpu.VMEM((FCAP,), jnp.int32),
            pltpu.VMEM((32,), jnp.int32),
            pltpu.VMEM((32,), jnp.int32),
            pltpu.VMEM((32,), jnp.int32),
            pltpu.VMEM((32, H), jnp.float32),
            pltpu.VMEM((32, H), jnp.float32),
            pltpu.SemaphoreType.DMA,
            pltpu.SemaphoreType.DMA,
        ],
        compiler_params=pltpu.CompilerParams(needs_layout_passes=False),
    )(h, src, dst)


def _finish(out, gs, b):
    nrm = jnp.sqrt(jnp.sum(out * out, axis=1, keepdims=True))
    out = out / jnp.maximum(nrm, 1e-12)
    out = out * gs + b
    return jnp.where(out > 0, out, jnp.exp(jnp.minimum(out, 0.0)) - 1.0)


def _dense_body(mode, agg_ref, h_ref, rcnt_ref, wl_ref, bl_ref, wr_ref,
                gs_ref, b_ref, *rest):
    if mode == 0:
        out_ref, rcnt_out = rest[-2], rest[-1]
        cnt = agg_ref[:, 10:11]
        rcnt = 1.0 / jnp.maximum(cnt, 1.0)
        rcnt_out[...] = rcnt
    else:
        out_ref = rest[-1]
        rcnt = rcnt_ref[...]
    mean = agg_ref[...] * rcnt
    out = (jnp.dot(mean, wl_ref[...], preferred_element_type=jnp.float32)
           + jnp.dot(h_ref[...], wr_ref[...], preferred_element_type=jnp.float32)
           + bl_ref[...])
    out = _finish(out, gs_ref[...], b_ref[...])
    if mode == 2:
        (w0, b0, gs0, be0, w1, b1, gs1, be1, w2, b2) = rest[:-1]
        out = jnp.dot(out, w0[...], preferred_element_type=jnp.float32) + b0[...]
        out = jnp.maximum(out * gs0[...] + be0[...], 0.0)
        out = jnp.dot(out, w1[...], preferred_element_type=jnp.float32) + b1[...]
        out = jnp.maximum(out * gs1[...] + be1[...], 0.0)
        out = jnp.dot(out, w2[...], preferred_element_type=jnp.float32) + b2[...]
    out_ref[...] = out


def _dense(mode, agg, h, rcnt, wl, bl, wr, gs, b, head=()):
    row = lambda i: (i, 0)
    full = lambda i: (0, 0)
    args = [agg, h, rcnt, wl, bl, wr, gs, b] + list(head)
    in_specs = [
        pl.BlockSpec((BN, H), row),
        pl.BlockSpec((BN, H), row),
        pl.BlockSpec((BN, 1), row),
        pl.BlockSpec((H, H), full),
        pl.BlockSpec((1, H), full),
        pl.BlockSpec((H, H), full),
        pl.BlockSpec((1, H), full),
        pl.BlockSpec((1, H), full),
    ] + [pl.BlockSpec((H, H), full) if a.shape == (H, H)
         else pl.BlockSpec((1, H), full) for a in head]
    out_specs = [pl.BlockSpec((BN, H), row)]
    out_shape = [jax.ShapeDtypeStruct((NP, H), jnp.float32)]
    if mode == 0:
        out_specs.append(pl.BlockSpec((BN, 1), row))
        out_shape.append(jax.ShapeDtypeStruct((NP, 1), jnp.float32))
    res = pl.pallas_call(
        functools.partial(_dense_body, mode),
        grid=(GRID,),
        in_specs=in_specs,
        out_specs=out_specs,
        out_shape=out_shape,
    )(*args)
    return res if mode == 0 else res[0]


def kernel(x, edge_index, params):
    src = edge_index[0]
    dst = edge_index[1]
    h0 = jnp.zeros((NP, H), jnp.float32).at[:N, :10].set(x).at[:, 10].set(1.0)

    def r(v):
        return jnp.reshape(v, (1, H))

    def wpad(w):
        return jnp.zeros((H, H), jnp.float32).at[:w.shape[0], :w.shape[1]].set(w)

    sc = 1.0 / jnp.sqrt(1.0 + BN_EPS)
    rcnt0 = jnp.zeros((NP, 1), jnp.float32)

    agg = _agg128(h0, src, dst)
    h, rcnt = _dense(0, agg, h0, rcnt0,
                     wpad(params["conv0_Wl"]), r(params["conv0_bl"]),
                     wpad(params["conv0_Wr"]),
                     r(params["bn0_g"] * sc), r(params["bn0_b"]))

    head = (params["p0_W"], r(params["p0_b"]), r(params["p0_g"] * sc),
            r(params["p0_be"]),
            params["p1_W"], r(params["p1_b"]), r(params["p1_g"] * sc),
            r(params["p1_be"]),
            wpad(params["p2_W"]),
            jnp.zeros((1, H), jnp.float32).at[0, :OUT].set(params["p2_b"]))

    for i in range(1, 5):
        agg = _agg128(h, src, dst)
        h = _dense(2 if i == 4 else 1, agg, h, rcnt,
                   params[f"conv{i}_Wl"], r(params[f"conv{i}_bl"]),
                   params[f"conv{i}_Wr"],
                   r(params[f"bn{i}_g"] * sc), r(params[f"bn{i}_b"]),
                   head=head if i == 4 else ())
    return h[:N, :OUT]


# speedup vs baseline: 2.0815x; 2.0815x over previous
import functools

import jax
import jax.numpy as jnp
from jax import lax
from jax.experimental import pallas as pl
from jax.experimental.pallas import tpu as pltpu
from jax.experimental.pallas import tpu_sc as plsc

N = 100000
NP = 100096
E = 3200000
H = 128
OUT = 3
BN_EPS = 1e-5
BN = 3128
GRID = NP // BN

NC, NS = 2, 16
CHN = 12512
ACC = 12544
DUMP = 12512
NPA = NP
BE = 2000
NBLK = (E // NS) // BE
FG = 64
FCAP = BE + 2 * FG
_MESH = plsc.VectorSubcoreMesh(core_axis_name="c", subcore_axis_name="s")


def _sc_agg_body(h_hbm, src_hbm, dst_hbm, agg_hbm,
                 acc, sbuf, dbuf, csrcf, cdstf, g64s, g64d, rows):
    c = lax.axis_index("c")
    s = lax.axis_index("s")

    ept = E // NS
    ebase = s * ept
    zv = jnp.zeros((16,), jnp.int32)

    for j in range(4):
        chunk = 2 * j + c
        lo = chunk * CHN

        @pl.loop(0, FG)
        def _zr(i):
            for q in range(8):
                rows[i, pl.ds(q * 16, 16)] = jnp.zeros((16,), jnp.float32)

        @pl.when(s < 8)
        def _():
            @pl.loop(0, 1568 // FG)
            def _zc(k):
                pltpu.sync_copy(rows, acc.at[pl.ds(s * 1568 + k * FG, FG)])
            pltpu.sync_copy(rows.at[pl.ds(0, 1568 % FG)],
                            acc.at[pl.ds(s * 1568 + 1568 - 1568 % FG,
                                         1568 % FG)])
        plsc.subcore_barrier()

        @pl.loop(0, NBLK)
        def _blk(i):
            off = ebase + i * BE
            pltpu.sync_copy(src_hbm.at[pl.ds(off, BE)], sbuf)
            pltpu.sync_copy(dst_hbm.at[pl.ds(off, BE)], dbuf)

            @pl.loop(0, BE // 16, init_carry=jnp.int32(0))
            def _vec(v, cur):
                sv = sbuf[pl.ds(v * 16, 16)]
                dv = dbuf[pl.ds(v * 16, 16)]
                m = (dv >= lo) & (dv < lo + CHN)
                plsc.store_compressed(csrcf.at[pl.ds(cur, 16)], sv, mask=m)
                plsc.store_compressed(cdstf.at[pl.ds(cur, 16)], dv - lo,
                                      mask=m)
                return cur + jnp.sum(jnp.where(m, 1, 0).astype(jnp.int32))

            cur = _vec
            for t in range(FG // 16):
                csrcf[pl.ds(cur + 16 * t, 16)] = zv
                cdstf[pl.ds(cur + 16 * t, 16)] = zv + DUMP

            @pl.loop(0, (cur + FG - 1) // FG)
            def _flush(q):
                for t in range(FG // 16):
                    g64s[pl.ds(t * 16, 16)] = csrcf[pl.ds(q * FG + t * 16, 16)]
                    g64d[pl.ds(t * 16, 16)] = cdstf[pl.ds(q * FG + t * 16, 16)]
                pltpu.sync_copy(h_hbm.at[g64s], rows)
                pltpu.sync_copy(rows, acc.at[g64d], add=True)

        plsc.subcore_barrier()

        @pl.when(s < 7)
        def _():
            pltpu.sync_copy(acc.at[pl.ds(s * 1568, 1568)],
                            agg_hbm.at[pl.ds(lo + s * 1568, 1568)])
        @pl.when(s == 7)
        def _():
            pltpu.sync_copy(acc.at[pl.ds(7 * 1568, 1536)],
                            agg_hbm.at[pl.ds(lo + 7 * 1568, 1536)])
        plsc.subcore_barrier()


def _agg128(h, src, dst):
    return pl.kernel(
        _sc_agg_body,
        out_type=jax.ShapeDtypeStruct((NPA, H), jnp.float32),
        mesh=_MESH,
        scratch_types=[
            pltpu.VMEM_SHARED((ACC, H), jnp.float32),
            pltpu.VMEM((BE,), jnp.int32),
            pltpu.VMEM((BE,), jnp.int32),
            pltpu.VMEM((FCAP,), jnp.int32),
            pltpu.VMEM((FCAP,), jnp.int32),
            pltpu.VMEM((FG,), jnp.int32),
            pltpu.VMEM((FG,), jnp.int32),
            pltpu.VMEM((FG, H), jnp.float32),
        ],
        compiler_params=pltpu.CompilerParams(needs_layout_passes=False),
    )(h, src, dst)


def _finish(out, gs, b):
    nrm = jnp.sqrt(jnp.sum(out * out, axis=1, keepdims=True))
    out = out / jnp.maximum(nrm, 1e-12)
    out = out * gs + b
    return jnp.where(out > 0, out, jnp.exp(jnp.minimum(out, 0.0)) - 1.0)


def _dense_body(mode, agg_ref, h_ref, rcnt_ref, wl_ref, bl_ref, wr_ref,
                gs_ref, b_ref, *rest):
    if mode == 0:
        out_ref, rcnt_out = rest[-2], rest[-1]
        cnt = agg_ref[:, 10:11]
        rcnt = 1.0 / jnp.maximum(cnt, 1.0)
        rcnt_out[...] = rcnt
    else:
        out_ref = rest[-1]
        rcnt = rcnt_ref[...]
    mean = agg_ref[...] * rcnt
    out = (jnp.dot(mean, wl_ref[...], preferred_element_type=jnp.float32)
           + jnp.dot(h_ref[...], wr_ref[...], preferred_element_type=jnp.float32)
           + bl_ref[...])
    out = _finish(out, gs_ref[...], b_ref[...])
    if mode == 2:
        (w0, b0, gs0, be0, w1, b1, gs1, be1, w2, b2) = rest[:-1]
        out = jnp.dot(out, w0[...], preferred_element_type=jnp.float32) + b0[...]
        out = jnp.maximum(out * gs0[...] + be0[...], 0.0)
        out = jnp.dot(out, w1[...], preferred_element_type=jnp.float32) + b1[...]
        out = jnp.maximum(out * gs1[...] + be1[...], 0.0)
        out = jnp.dot(out, w2[...], preferred_element_type=jnp.float32) + b2[...]
    out_ref[...] = out


def _dense(mode, agg, h, rcnt, wl, bl, wr, gs, b, head=()):
    row = lambda i: (i, 0)
    full = lambda i: (0, 0)
    args = [agg, h, rcnt, wl, bl, wr, gs, b] + list(head)
    in_specs = [
        pl.BlockSpec((BN, H), row),
        pl.BlockSpec((BN, H), row),
        pl.BlockSpec((BN, 1), row),
        pl.BlockSpec((H, H), full),
        pl.BlockSpec((1, H), full),
        pl.BlockSpec((H, H), full),
        pl.BlockSpec((1, H), full),
        pl.BlockSpec((1, H), full),
    ] + [pl.BlockSpec((H, H), full) if a.shape == (H, H)
         else pl.BlockSpec((1, H), full) for a in head]
    out_specs = [pl.BlockSpec((BN, H), row)]
    out_shape = [jax.ShapeDtypeStruct((NP, H), jnp.float32)]
    if mode == 0:
        out_specs.append(pl.BlockSpec((BN, 1), row))
        out_shape.append(jax.ShapeDtypeStruct((NP, 1), jnp.float32))
    res = pl.pallas_call(
        functools.partial(_dense_body, mode),
        grid=(GRID,),
        in_specs=in_specs,
        out_specs=out_specs,
        out_shape=out_shape,
    )(*args)
    return res if mode == 0 else res[0]


def kernel(x, edge_index, params):
    src = edge_index[0]
    dst = edge_index[1]
    h0 = jnp.zeros((NP, H), jnp.float32).at[:N, :10].set(x).at[:, 10].set(1.0)

    def r(v):
        return jnp.reshape(v, (1, H))

    def wpad(w):
        return jnp.zeros((H, H), jnp.float32).at[:w.shape[0], :w.shape[1]].set(w)

    sc = 1.0 / jnp.sqrt(1.0 + BN_EPS)
    rcnt0 = jnp.zeros((NP, 1), jnp.float32)

    agg = _agg128(h0, src, dst)
    h, rcnt = _dense(0, agg, h0, rcnt0,
                     wpad(params["conv0_Wl"]), r(params["conv0_bl"]),
                     wpad(params["conv0_Wr"]),
                     r(params["bn0_g"] * sc), r(params["bn0_b"]))

    head = (params["p0_W"], r(params["p0_b"]), r(params["p0_g"] * sc),
            r(params["p0_be"]),
            params["p1_W"], r(params["p1_b"]), r(params["p1_g"] * sc),
            r(params["p1_be"]),
            wpad(params["p2_W"]),
            jnp.zeros((1, H), jnp.float32).at[0, :OUT].set(params["p2_b"]))

    for i in range(1, 5):
        agg = _agg128(h, src, dst)
        h = _dense(2 if i == 4 else 1, agg, h, rcnt,
                   params[f"conv{i}_Wl"], r(params[f"conv{i}_bl"]),
                   params[f"conv{i}_Wr"],
                   r(params[f"bn{i}_g"] * sc), r(params[f"bn{i}_b"]),
                   head=head if i == 4 else ())
    return h[:N, :OUT]


# speedup vs baseline: 3.4706x; 1.6674x over previous
import functools

import jax
import jax.numpy as jnp
from jax import lax
from jax.experimental import pallas as pl
from jax.experimental.pallas import tpu as pltpu
from jax.experimental.pallas import tpu_sc as plsc

N = 100000
NP = 100096
E = 3200000
H = 128
OUT = 3
BN_EPS = 1e-5
BN = 3128
GRID = NP // BN

NC, NS = 2, 16
CHN = 12512
ACC = 12544
DUMP = 12512
NPA = NP
BE = 2000
NBLK = (E // NS) // BE
FG = 32
FCAP = BE + 2 * FG
_MESH = plsc.VectorSubcoreMesh(core_axis_name="c", subcore_axis_name="s")


def _sc_agg_body(h_hbm, src_hbm, dst_hbm, agg_hbm,
                 acc, sbuf, dbuf, csrcf, cdstf, g64s, g64d, rows):
    c = lax.axis_index("c")
    s = lax.axis_index("s")

    ept = E // NS
    ebase = s * ept
    zv = jnp.zeros((16,), jnp.int32)

    for j in range(4):
        chunk = 2 * j + c
        lo = chunk * CHN

        @pl.loop(0, FG)
        def _zr(i):
            for q in range(8):
                rows[i, pl.ds(q * 16, 16)] = jnp.zeros((16,), jnp.float32)

        @pl.when(s < 8)
        def _():
            @pl.loop(0, 1568 // FG)
            def _zc(k):
                pltpu.sync_copy(rows, acc.at[pl.ds(s * 1568 + k * FG, FG)])
            pltpu.sync_copy(rows.at[pl.ds(0, 1568 % FG)],
                            acc.at[pl.ds(s * 1568 + 1568 - 1568 % FG,
                                         1568 % FG)])
        plsc.subcore_barrier()

        @pl.loop(0, NBLK)
        def _blk(i):
            off = ebase + i * BE
            pltpu.sync_copy(src_hbm.at[pl.ds(off, BE)], sbuf)
            pltpu.sync_copy(dst_hbm.at[pl.ds(off, BE)], dbuf)

            @pl.loop(0, BE // 16, init_carry=jnp.int32(0))
            def _vec(v, cur):
                sv = sbuf[pl.ds(v * 16, 16)]
                dv = dbuf[pl.ds(v * 16, 16)]
                m = (dv >= lo) & (dv < lo + CHN)
                plsc.store_compressed(csrcf.at[pl.ds(cur, 16)], sv, mask=m)
                plsc.store_compressed(cdstf.at[pl.ds(cur, 16)], dv - lo,
                                      mask=m)
                return cur + jnp.sum(jnp.where(m, 1, 0).astype(jnp.int32))

            cur = _vec
            iota = lax.iota(jnp.int32, 16)
            for t in range(FG // 16):
                csrcf[pl.ds(cur + 16 * t, 16)] = zv
                cdstf[pl.ds(cur + 16 * t, 16)] = iota + (DUMP + 16 * (t % 2))

            @pl.loop(0, (cur + FG - 1) // FG)
            def _flush(q):
                for t in range(FG // 16):
                    g64s[pl.ds(t * 16, 16)] = csrcf[pl.ds(q * FG + t * 16, 16)]
                    g64d[pl.ds(t * 16, 16)] = cdstf[pl.ds(q * FG + t * 16, 16)]
                pltpu.sync_copy(h_hbm.at[g64s], rows)
                pltpu.sync_copy(rows, acc.at[g64d], add=True)

        plsc.subcore_barrier()

        @pl.when(s < 7)
        def _():
            pltpu.sync_copy(acc.at[pl.ds(s * 1568, 1568)],
                            agg_hbm.at[pl.ds(lo + s * 1568, 1568)])
        @pl.when(s == 7)
        def _():
            pltpu.sync_copy(acc.at[pl.ds(7 * 1568, 1536)],
                            agg_hbm.at[pl.ds(lo + 7 * 1568, 1536)])
        plsc.subcore_barrier()


def _agg128(h, src, dst):
    return pl.kernel(
        _sc_agg_body,
        out_type=jax.ShapeDtypeStruct((NPA, H), jnp.float32),
        mesh=_MESH,
        scratch_types=[
            pltpu.VMEM_SHARED((ACC, H), jnp.float32),
            pltpu.VMEM((BE,), jnp.int32),
            pltpu.VMEM((BE,), jnp.int32),
            pltpu.VMEM((FCAP,), jnp.int32),
            pltpu.VMEM((FCAP,), jnp.int32),
            pltpu.VMEM((FG,), jnp.int32),
            pltpu.VMEM((FG,), jnp.int32),
            pltpu.VMEM((FG, H), jnp.float32),
        ],
        compiler_params=pltpu.CompilerParams(needs_layout_passes=False),
    )(h, src, dst)


def _finish(out, gs, b):
    nrm = jnp.sqrt(jnp.sum(out * out, axis=1, keepdims=True))
    out = out / jnp.maximum(nrm, 1e-12)
    out = out * gs + b
    return jnp.where(out > 0, out, jnp.exp(jnp.minimum(out, 0.0)) - 1.0)


def _dense_body(mode, agg_ref, h_ref, rcnt_ref, wl_ref, bl_ref, wr_ref,
                gs_ref, b_ref, *rest):
    if mode == 0:
        out_ref, rcnt_out = rest[-2], rest[-1]
        cnt = agg_ref[:, 10:11]
        rcnt = 1.0 / jnp.maximum(cnt, 1.0)
        rcnt_out[...] = rcnt
    else:
        out_ref = rest[-1]
        rcnt = rcnt_ref[...]
    mean = agg_ref[...] * rcnt
    out = (jnp.dot(mean, wl_ref[...], preferred_element_type=jnp.float32)
           + jnp.dot(h_ref[...], wr_ref[...], preferred_element_type=jnp.float32)
           + bl_ref[...])
    out = _finish(out, gs_ref[...], b_ref[...])
    if mode == 2:
        (w0, b0, gs0, be0, w1, b1, gs1, be1, w2, b2) = rest[:-1]
        out = jnp.dot(out, w0[...], preferred_element_type=jnp.float32) + b0[...]
        out = jnp.maximum(out * gs0[...] + be0[...], 0.0)
        out = jnp.dot(out, w1[...], preferred_element_type=jnp.float32) + b1[...]
        out = jnp.maximum(out * gs1[...] + be1[...], 0.0)
        out = jnp.dot(out, w2[...], preferred_element_type=jnp.float32) + b2[...]
    out_ref[...] = out


def _dense(mode, agg, h, rcnt, wl, bl, wr, gs, b, head=()):
    row = lambda i: (i, 0)
    full = lambda i: (0, 0)
    args = [agg, h, rcnt, wl, bl, wr, gs, b] + list(head)
    in_specs = [
        pl.BlockSpec((BN, H), row),
        pl.BlockSpec((BN, H), row),
        pl.BlockSpec((BN, 1), row),
        pl.BlockSpec((H, H), full),
        pl.BlockSpec((1, H), full),
        pl.BlockSpec((H, H), full),
        pl.BlockSpec((1, H), full),
        pl.BlockSpec((1, H), full),
    ] + [pl.BlockSpec((H, H), full) if a.shape == (H, H)
         else pl.BlockSpec((1, H), full) for a in head]
    out_specs = [pl.BlockSpec((BN, H), row)]
    out_shape = [jax.ShapeDtypeStruct((NP, H), jnp.float32)]
    if mode == 0:
        out_specs.append(pl.BlockSpec((BN, 1), row))
        out_shape.append(jax.ShapeDtypeStruct((NP, 1), jnp.float32))
    res = pl.pallas_call(
        functools.partial(_dense_body, mode),
        grid=(GRID,),
        in_specs=in_specs,
        out_specs=out_specs,
        out_shape=out_shape,
    )(*args)
    return res if mode == 0 else res[0]


def kernel(x, edge_index, params):
    src = edge_index[0]
    dst = edge_index[1]
    h0 = jnp.zeros((NP, H), jnp.float32).at[:N, :10].set(x).at[:, 10].set(1.0)

    def r(v):
        return jnp.reshape(v, (1, H))

    def wpad(w):
        return jnp.zeros((H, H), jnp.float32).at[:w.shape[0], :w.shape[1]].set(w)

    sc = 1.0 / jnp.sqrt(1.0 + BN_EPS)
    rcnt0 = jnp.zeros((NP, 1), jnp.float32)

    agg = _agg128(h0, src, dst)
    h, rcnt = _dense(0, agg, h0, rcnt0,
                     wpad(params["conv0_Wl"]), r(params["conv0_bl"]),
                     wpad(params["conv0_Wr"]),
                     r(params["bn0_g"] * sc), r(params["bn0_b"]))

    head = (params["p0_W"], r(params["p0_b"]), r(params["p0_g"] * sc),
            r(params["p0_be"]),
            params["p1_W"], r(params["p1_b"]), r(params["p1_g"] * sc),
            r(params["p1_be"]),
            wpad(params["p2_W"]),
            jnp.zeros((1, H), jnp.float32).at[0, :OUT].set(params["p2_b"]))

    for i in range(1, 5):
        agg = _agg128(h, src, dst)
        h = _dense(2 if i == 4 else 1, agg, h, rcnt,
                   params[f"conv{i}_Wl"], r(params[f"conv{i}_bl"]),
                   params[f"conv{i}_Wr"],
                   r(params[f"bn{i}_g"] * sc), r(params[f"bn{i}_b"]),
                   head=head if i == 4 else ())
    return h[:N, :OUT]


# speedup vs baseline: 5.6349x; 1.6236x over previous
import functools

import jax
import jax.numpy as jnp
from jax import lax
from jax.experimental import pallas as pl
from jax.experimental.pallas import tpu as pltpu
from jax.experimental.pallas import tpu_sc as plsc

N = 100000
NP = 100096
E = 3200000
H = 128
OUT = 3
BN_EPS = 1e-5
BN = 3128
GRID = NP // BN

NC, NS = 2, 16
CHN = 12512
ACC = 12544
DUMP = 12512
NPA = NP
BE = 4000
NBLK = (E // NS) // BE
FG = 32
FCAP = BE + 2 * FG
_MESH = plsc.VectorSubcoreMesh(core_axis_name="c", subcore_axis_name="s")


def _sc_agg_body(h_hbm, src_hbm, dst_hbm, agg_hbm,
                 acc, sbuf, dbuf, csrcf, cdstf, g64s, g64d, rows):
    c = lax.axis_index("c")
    s = lax.axis_index("s")

    ept = E // NS
    ebase = s * ept
    zv = jnp.zeros((16,), jnp.int32)

    for j in range(4):
        chunk = 2 * j + c
        lo = chunk * CHN

        @pl.loop(0, FG)
        def _zr(i):
            for q in range(8):
                rows[i, pl.ds(q * 16, 16)] = jnp.zeros((16,), jnp.float32)

        @pl.when(s < 8)
        def _():
            @pl.loop(0, 1568 // FG)
            def _zc(k):
                pltpu.sync_copy(rows, acc.at[pl.ds(s * 1568 + k * FG, FG)])
            pltpu.sync_copy(rows.at[pl.ds(0, 1568 % FG)],
                            acc.at[pl.ds(s * 1568 + 1568 - 1568 % FG,
                                         1568 % FG)])
        plsc.subcore_barrier()

        @pl.loop(0, NBLK)
        def _blk(i):
            off = ebase + i * BE
            pltpu.sync_copy(src_hbm.at[pl.ds(off, BE)], sbuf)
            pltpu.sync_copy(dst_hbm.at[pl.ds(off, BE)], dbuf)

            @pl.loop(0, BE // 16, init_carry=jnp.int32(0))
            def _vec(v, cur):
                sv = sbuf[pl.ds(v * 16, 16)]
                dv = dbuf[pl.ds(v * 16, 16)]
                m = (dv >= lo) & (dv < lo + CHN)
                plsc.store_compressed(csrcf.at[pl.ds(cur, 16)], sv, mask=m)
                plsc.store_compressed(cdstf.at[pl.ds(cur, 16)], dv - lo,
                                      mask=m)
                return cur + jnp.sum(jnp.where(m, 1, 0).astype(jnp.int32))

            cur = _vec
            iota = lax.iota(jnp.int32, 16)
            for t in range(FG // 16):
                csrcf[pl.ds(cur + 16 * t, 16)] = zv
                cdstf[pl.ds(cur + 16 * t, 16)] = iota + (DUMP + 16 * (t % 2))

            @pl.loop(0, (cur + FG - 1) // FG)
            def _flush(q):
                for t in range(FG // 16):
                    g64s[pl.ds(t * 16, 16)] = csrcf[pl.ds(q * FG + t * 16, 16)]
                    g64d[pl.ds(t * 16, 16)] = cdstf[pl.ds(q * FG + t * 16, 16)]
                pltpu.sync_copy(h_hbm.at[g64s], rows)
                pltpu.sync_copy(rows, acc.at[g64d], add=True)

        plsc.subcore_barrier()

        @pl.when(s < 7)
        def _():
            pltpu.sync_copy(acc.at[pl.ds(s * 1568, 1568)],
                            agg_hbm.at[pl.ds(lo + s * 1568, 1568)])
        @pl.when(s == 7)
        def _():
            pltpu.sync_copy(acc.at[pl.ds(7 * 1568, 1536)],
                            agg_hbm.at[pl.ds(lo + 7 * 1568, 1536)])
        plsc.subcore_barrier()


def _agg128(h, src, dst):
    return pl.kernel(
        _sc_agg_body,
        out_type=jax.ShapeDtypeStruct((NPA, H), jnp.float32),
        mesh=_MESH,
        scratch_types=[
            pltpu.VMEM_SHARED((ACC, H), jnp.float32),
            pltpu.VMEM((BE,), jnp.int32),
            pltpu.VMEM((BE,), jnp.int32),
            pltpu.VMEM((FCAP,), jnp.int32),
            pltpu.VMEM((FCAP,), jnp.int32),
            pltpu.VMEM((FG,), jnp.int32),
            pltpu.VMEM((FG,), jnp.int32),
            pltpu.VMEM((FG, H), jnp.float32),
        ],
        compiler_params=pltpu.CompilerParams(needs_layout_passes=False),
    )(h, src, dst)


def _finish(out, gs, b):
    nrm = jnp.sqrt(jnp.sum(out * out, axis=1, keepdims=True))
    out = out / jnp.maximum(nrm, 1e-12)
    out = out * gs + b
    return jnp.where(out > 0, out, jnp.exp(jnp.minimum(out, 0.0)) - 1.0)


def _dense_body(mode, agg_ref, h_ref, rcnt_ref, wl_ref, bl_ref, wr_ref,
                gs_ref, b_ref, *rest):
    if mode == 0:
        out_ref, rcnt_out = rest[-2], rest[-1]
        cnt = agg_ref[:, 10:11]
        rcnt = 1.0 / jnp.maximum(cnt, 1.0)
        rcnt_out[...] = rcnt
    else:
        out_ref = rest[-1]
        rcnt = rcnt_ref[...]
    mean = agg_ref[...] * rcnt
    out = (jnp.dot(mean, wl_ref[...], preferred_element_type=jnp.float32)
           + jnp.dot(h_ref[...], wr_ref[...], preferred_element_type=jnp.float32)
           + bl_ref[...])
    out = _finish(out, gs_ref[...], b_ref[...])
    if mode == 2:
        (w0, b0, gs0, be0, w1, b1, gs1, be1, w2, b2) = rest[:-1]
        out = jnp.dot(out, w0[...], preferred_element_type=jnp.float32) + b0[...]
        out = jnp.maximum(out * gs0[...] + be0[...], 0.0)
        out = jnp.dot(out, w1[...], preferred_element_type=jnp.float32) + b1[...]
        out = jnp.maximum(out * gs1[...] + be1[...], 0.0)
        out = jnp.dot(out, w2[...], preferred_element_type=jnp.float32) + b2[...]
    out_ref[...] = out


def _dense(mode, agg, h, rcnt, wl, bl, wr, gs, b, head=()):
    row = lambda i: (i, 0)
    full = lambda i: (0, 0)
    args = [agg, h, rcnt, wl, bl, wr, gs, b] + list(head)
    in_specs = [
        pl.BlockSpec((BN, H), row),
        pl.BlockSpec((BN, H), row),
        pl.BlockSpec((BN, 1), row),
        pl.BlockSpec((H, H), full),
        pl.BlockSpec((1, H), full),
        pl.BlockSpec((H, H), full),
        pl.BlockSpec((1, H), full),
        pl.BlockSpec((1, H), full),
    ] + [pl.BlockSpec((H, H), full) if a.shape == (H, H)
         else pl.BlockSpec((1, H), full) for a in head]
    out_specs = [pl.BlockSpec((BN, H), row)]
    out_shape = [jax.ShapeDtypeStruct((NP, H), jnp.float32)]
    if mode == 0:
        out_specs.append(pl.BlockSpec((BN, 1), row))
        out_shape.append(jax.ShapeDtypeStruct((NP, 1), jnp.float32))
    res = pl.pallas_call(
        functools.partial(_dense_body, mode),
        grid=(GRID,),
        in_specs=in_specs,
        out_specs=out_specs,
        out_shape=out_shape,
    )(*args)
    return res if mode == 0 else res[0]


def kernel(x, edge_index, params):
    src = edge_index[0]
    dst = edge_index[1]
    h0 = jnp.zeros((NP, H), jnp.float32).at[:N, :10].set(x).at[:, 10].set(1.0)

    def r(v):
        return jnp.reshape(v, (1, H))

    def wpad(w):
        return jnp.zeros((H, H), jnp.float32).at[:w.shape[0], :w.shape[1]].set(w)

    sc = 1.0 / jnp.sqrt(1.0 + BN_EPS)
    rcnt0 = jnp.zeros((NP, 1), jnp.float32)

    agg = _agg128(h0, src, dst)
    h, rcnt = _dense(0, agg, h0, rcnt0,
                     wpad(params["conv0_Wl"]), r(params["conv0_bl"]),
                     wpad(params["conv0_Wr"]),
                     r(params["bn0_g"] * sc), r(params["bn0_b"]))

    head = (params["p0_W"], r(params["p0_b"]), r(params["p0_g"] * sc),
            r(params["p0_be"]),
            params["p1_W"], r(params["p1_b"]), r(params["p1_g"] * sc),
            r(params["p1_be"]),
            wpad(params["p2_W"]),
            jnp.zeros((1, H), jnp.float32).at[0, :OUT].set(params["p2_b"]))

    for i in range(1, 5):
        agg = _agg128(h, src, dst)
        h = _dense(2 if i == 4 else 1, agg, h, rcnt,
                   params[f"conv{i}_Wl"], r(params[f"conv{i}_bl"]),
                   params[f"conv{i}_Wr"],
                   r(params[f"bn{i}_g"] * sc), r(params[f"bn{i}_b"]),
                   head=head if i == 4 else ())
    return h[:N, :OUT]
